# baseline (speedup 1.0000x reference)
import jax
import jax.numpy as jnp
from jax.experimental import pallas as pl

N = 10000
N_STEPS = 2


def _gru(a, h, Wih, bih, Whh, bhh):
    gi = a @ Wih.T + bih
    gh = h @ Whh.T + bhh
    ir, iz, inew = jnp.split(gi, 3, axis=-1)
    hr, hz, hnew = jnp.split(gh, 3, axis=-1)
    r = jax.nn.sigmoid(ir + hr)
    z = jax.nn.sigmoid(iz + hz)
    n = jnp.tanh(inew + r * hnew)
    return (1.0 - z) * n + z * h


def _conv(x, src, dst, etype, Ws, bs, Wih, bih, Whh, bhh):
    h = x
    for _ in range(N_STEPS):
        Wh = jnp.einsum('kod,nd->kno', Ws, h) + bs[:, None, :]
        m = Wh[etype, src]
        a = jax.ops.segment_sum(m, dst, num_segments=N)
        h = _gru(a, h, Wih, bih, Whh, bhh)
    return h


def kernel(x, ast_edge_index, cpg_edge_index, cpg_etypes, ast_W, ast_b, ast_Wih, ast_bih, ast_Whh, ast_bhh, cpg_Ws, cpg_bs, cpg_Wih, cpg_bih, cpg_Whh, cpg_bhh, fn1_W, fn1_b, fn2_W, fn2_b):
    src_a, dst_a = ast_edge_index[0], ast_edge_index[1]
    h_ast = _conv(x, src_a, dst_a, jnp.zeros_like(src_a), ast_W[None], ast_b[None], ast_Wih, ast_bih, ast_Whh, ast_bhh)
    hiddens = jnp.concatenate([h_ast, x], axis=-1) @ fn1_W.T + fn1_b
    src_c, dst_c = cpg_edge_index[0], cpg_edge_index[1]
    h_cpg = _conv(hiddens, src_c, dst_c, cpg_etypes, cpg_Ws, cpg_bs, cpg_Wih, cpg_bih, cpg_Whh, cpg_bhh)
    logits = jnp.concatenate([h_cpg, hiddens], axis=-1) @ fn2_W.T + fn2_b
    return logits


# speedup vs baseline: 6.4942x; 6.4942x over previous
import functools

import jax
import jax.numpy as jnp
from jax import lax
from jax.experimental import pallas as pl
from jax.experimental.pallas import tpu as pltpu
from jax.experimental.pallas import tpu_sc as plsc

N = 10000
N_PAD = 10240
D = 256
H = 128
E = 160000
N_STEPS = 2
NS = 16
EB = 80
R = 1000


def _wh_body(h_ref, wt_ref, b_ref, lo_ref, hi_ref):
    wh = jnp.dot(h_ref[...], wt_ref[0], preferred_element_type=jnp.float32)
    wh = wh + b_ref[0]
    lo_ref[...] = wh[:, :H]
    hi_ref[...] = wh[:, H:]


def _wh_call(h, wT, b, K):
    nb = N // R
    return pl.pallas_call(
        _wh_body,
        grid=(K, nb),
        in_specs=[
            pl.BlockSpec((R, D), lambda k, i: (i, 0)),
            pl.BlockSpec((1, D, D), lambda k, i: (k, 0, 0)),
            pl.BlockSpec((1, 1, D), lambda k, i: (k, 0, 0)),
        ],
        out_specs=[
            pl.BlockSpec((R, H), lambda k, i: (k * nb + i, 0)),
            pl.BlockSpec((R, H), lambda k, i: (k * nb + i, 0)),
        ],
        out_shape=[
            jax.ShapeDtypeStruct((K * N, H), jnp.float32),
            jax.ShapeDtypeStruct((K * N, H), jnp.float32),
        ],
    )(h, wT, b)


def _gru_body(alo_ref, ahi_ref, h_ref, wihT_ref, bih_ref, whhT_ref, bhh_ref,
              out_ref):
    gi = jnp.dot(alo_ref[...], wihT_ref[:H], preferred_element_type=jnp.float32)
    gi = gi + jnp.dot(ahi_ref[...], wihT_ref[H:],
                      preferred_element_type=jnp.float32)
    gi = gi + bih_ref[...]
    h = h_ref[...]
    gh = jnp.dot(h, whhT_ref[...], preferred_element_type=jnp.float32)
    gh = gh + bhh_ref[...]
    r = jax.nn.sigmoid(gi[:, :D] + gh[:, :D])
    z = jax.nn.sigmoid(gi[:, D:2 * D] + gh[:, D:2 * D])
    n = jnp.tanh(gi[:, 2 * D:] + r * gh[:, 2 * D:])
    out_ref[...] = (1.0 - z) * n + z * h


def _gru_call(alo, ahi, h, wihT, bih, whhT, bhh):
    nb = N // R
    return pl.pallas_call(
        _gru_body,
        grid=(nb,),
        in_specs=[
            pl.BlockSpec((R, H), lambda i: (i, 0)),
            pl.BlockSpec((R, H), lambda i: (i, 0)),
            pl.BlockSpec((R, D), lambda i: (i, 0)),
            pl.BlockSpec((D, 3 * D), lambda i: (0, 0)),
            pl.BlockSpec((1, 3 * D), lambda i: (0, 0)),
            pl.BlockSpec((D, 3 * D), lambda i: (0, 0)),
            pl.BlockSpec((1, 3 * D), lambda i: (0, 0)),
        ],
        out_specs=pl.BlockSpec((R, D), lambda i: (i, 0)),
        out_shape=jax.ShapeDtypeStruct((N, D), jnp.float32),
    )(alo, ahi, h, wihT, bih, whhT, bhh)


def _lin2_body(a_ref, b_ref, wta_ref, wtb_ref, bias_ref, out_ref):
    out = jnp.dot(a_ref[...], wta_ref[...], preferred_element_type=jnp.float32)
    out = out + jnp.dot(b_ref[...], wtb_ref[...],
                        preferred_element_type=jnp.float32)
    out_ref[...] = out + bias_ref[...]


def _lin2_call(a, b, wta, wtb, bias, out_dim):
    nb = N // R
    return pl.pallas_call(
        _lin2_body,
        grid=(nb,),
        in_specs=[
            pl.BlockSpec((R, D), lambda i: (i, 0)),
            pl.BlockSpec((R, D), lambda i: (i, 0)),
            pl.BlockSpec((D, out_dim), lambda i: (0, 0)),
            pl.BlockSpec((D, out_dim), lambda i: (0, 0)),
            pl.BlockSpec((1, out_dim), lambda i: (0, 0)),
        ],
        out_specs=pl.BlockSpec((R, out_dim), lambda i: (i, 0)),
        out_shape=jax.ShapeDtypeStruct((N, out_dim), jnp.float32),
    )(a, b, wta, wtb, bias)


def _make_scatter(KN):
    mesh = plsc.VectorSubcoreMesh(core_axis_name="c", subcore_axis_name="s")
    ept = E // NS
    nbatch = ept // EB
    rpt = N_PAD // NS

    @functools.partial(
        pl.kernel,
        out_type=[
            jax.ShapeDtypeStruct((N_PAD, H), jnp.float32),
            jax.ShapeDtypeStruct((N_PAD, H), jnp.float32),
        ],
        mesh=mesh,
        scratch_types=[
            pltpu.VMEM((EB,), jnp.int32),
            pltpu.VMEM((EB,), jnp.int32),
            pltpu.VMEM((EB, H), jnp.float32),
            pltpu.VMEM_SHARED((N_PAD, H), jnp.float32),
            pltpu.SemaphoreType.DMA,
        ],
    )
    def scatter_k(wh_lo, wh_hi, gidx_hbm, dst_hbm, zeros_hbm, out_lo, out_hi,
                  gidx_v, dst_v, rows_v, acc, sem):
        c = lax.axis_index("c")
        s = lax.axis_index("s")
        pltpu.sync_copy(zeros_hbm.at[pl.ds(s * rpt, rpt)],
                        acc.at[pl.ds(s * rpt, rpt)])
        plsc.subcore_barrier()

        base0 = s * ept

        def body(i, _):
            base = base0 + i * EB
            pltpu.sync_copy(gidx_hbm.at[pl.ds(base, EB)], gidx_v)
            pltpu.sync_copy(dst_hbm.at[pl.ds(base, EB)], dst_v)

            @pl.when(c == 0)
            def _():
                pltpu.async_copy(wh_lo.at[gidx_v], rows_v, sem).wait()

            @pl.when(c == 1)
            def _():
                pltpu.async_copy(wh_hi.at[gidx_v], rows_v, sem).wait()

            pltpu.sync_copy(rows_v, acc.at[dst_v], add=True)
            return ()

        lax.fori_loop(0, nbatch, body, ())
        plsc.subcore_barrier()

        ob = s * rpt

        @pl.when(c == 0)
        def _():
            pltpu.sync_copy(acc.at[pl.ds(ob, rpt)], out_lo.at[pl.ds(ob, rpt)])

        @pl.when(c == 1)
        def _():
            pltpu.sync_copy(acc.at[pl.ds(ob, rpt)], out_hi.at[pl.ds(ob, rpt)])

    return scatter_k


_scatter_ast = _make_scatter(N)
_scatter_cpg = _make_scatter(3 * N)


def _conv(h, gidx, dst, zeros, wT, b, wihT, bih, whhT, bhh, scatter, K):
    for _ in range(N_STEPS):
        lo, hi = _wh_call(h, wT, b, K)
        alo, ahi = scatter(lo, hi, gidx, dst, zeros)
        h = _gru_call(alo[:N], ahi[:N], h, wihT, bih, whhT, bhh)
    return h


def kernel(x, ast_edge_index, cpg_edge_index, cpg_etypes, ast_W, ast_b,
           ast_Wih, ast_bih, ast_Whh, ast_bhh, cpg_Ws, cpg_bs, cpg_Wih,
           cpg_bih, cpg_Whh, cpg_bhh, fn1_W, fn1_b, fn2_W, fn2_b):
    zeros = jnp.zeros((N_PAD, H), jnp.float32)

    gidx_a = ast_edge_index[0].astype(jnp.int32)
    dst_a = ast_edge_index[1].astype(jnp.int32)
    h_ast = _conv(
        x, gidx_a, dst_a, zeros,
        jnp.transpose(ast_W)[None], ast_b[None, None],
        jnp.transpose(ast_Wih), ast_bih[None], jnp.transpose(ast_Whh),
        ast_bhh[None], _scatter_ast, 1)

    fn1_WT = jnp.transpose(fn1_W)
    hiddens = _lin2_call(h_ast, x, fn1_WT[:D], fn1_WT[D:], fn1_b[None], D)

    gidx_c = (cpg_etypes.astype(jnp.int32) * N
              + cpg_edge_index[0].astype(jnp.int32))
    dst_c = cpg_edge_index[1].astype(jnp.int32)
    h_cpg = _conv(
        hiddens, gidx_c, dst_c, zeros,
        jnp.transpose(cpg_Ws, (0, 2, 1)), cpg_bs[:, None],
        jnp.transpose(cpg_Wih), cpg_bih[None], jnp.transpose(cpg_Whh),
        cpg_bhh[None], _scatter_cpg, 3)

    fn2_WT = jnp.transpose(fn2_W)
    logits = _lin2_call(h_cpg, hiddens, fn2_WT[:D], fn2_WT[D:], fn2_b[None], D)
    return logits


# speedup vs baseline: 10.9809x; 1.6909x over previous
import functools

import jax
import jax.numpy as jnp
from jax import lax
from jax.experimental import pallas as pl
from jax.experimental.pallas import tpu as pltpu
from jax.experimental.pallas import tpu_sc as plsc

N = 10000
N_PAD = 10240
D = 256
H = 128
E = 160000
N_STEPS = 2
NS = 16
EB = 80
R = 1000


def _wh_body(h_ref, wt_ref, b_ref, lo_ref, hi_ref):
    wh = jnp.dot(h_ref[...], wt_ref[0], preferred_element_type=jnp.float32)
    wh = wh + b_ref[0]
    lo_ref[...] = wh[:, :H]
    hi_ref[...] = wh[:, H:]


def _wh_call(h, wT, b, K):
    nb = N // R
    return pl.pallas_call(
        _wh_body,
        grid=(K, nb),
        in_specs=[
            pl.BlockSpec((R, D), lambda k, i: (i, 0)),
            pl.BlockSpec((1, D, D), lambda k, i: (k, 0, 0)),
            pl.BlockSpec((1, 1, D), lambda k, i: (k, 0, 0)),
        ],
        out_specs=[
            pl.BlockSpec((R, H), lambda k, i: (k * nb + i, 0)),
            pl.BlockSpec((R, H), lambda k, i: (k * nb + i, 0)),
        ],
        out_shape=[
            jax.ShapeDtypeStruct((K * N, H), jnp.float32),
            jax.ShapeDtypeStruct((K * N, H), jnp.float32),
        ],
    )(h, wT, b)


def _gru_body(alo_ref, ahi_ref, h_ref, wihT_ref, bih_ref, whhT_ref, bhh_ref,
              out_ref):
    gi = jnp.dot(alo_ref[...], wihT_ref[:H], preferred_element_type=jnp.float32)
    gi = gi + jnp.dot(ahi_ref[...], wihT_ref[H:],
                      preferred_element_type=jnp.float32)
    gi = gi + bih_ref[...]
    h = h_ref[...]
    gh = jnp.dot(h, whhT_ref[...], preferred_element_type=jnp.float32)
    gh = gh + bhh_ref[...]
    r = jax.nn.sigmoid(gi[:, :D] + gh[:, :D])
    z = jax.nn.sigmoid(gi[:, D:2 * D] + gh[:, D:2 * D])
    n = jnp.tanh(gi[:, 2 * D:] + r * gh[:, 2 * D:])
    out_ref[...] = (1.0 - z) * n + z * h


def _gru_call(alo, ahi, h, wihT, bih, whhT, bhh):
    nb = N // R
    return pl.pallas_call(
        _gru_body,
        grid=(nb,),
        in_specs=[
            pl.BlockSpec((R, H), lambda i: (i, 0)),
            pl.BlockSpec((R, H), lambda i: (i, 0)),
            pl.BlockSpec((R, D), lambda i: (i, 0)),
            pl.BlockSpec((D, 3 * D), lambda i: (0, 0)),
            pl.BlockSpec((1, 3 * D), lambda i: (0, 0)),
            pl.BlockSpec((D, 3 * D), lambda i: (0, 0)),
            pl.BlockSpec((1, 3 * D), lambda i: (0, 0)),
        ],
        out_specs=pl.BlockSpec((R, D), lambda i: (i, 0)),
        out_shape=jax.ShapeDtypeStruct((N, D), jnp.float32),
    )(alo, ahi, h, wihT, bih, whhT, bhh)


def _lin2_body(a_ref, b_ref, wta_ref, wtb_ref, bias_ref, out_ref):
    out = jnp.dot(a_ref[...], wta_ref[...], preferred_element_type=jnp.float32)
    out = out + jnp.dot(b_ref[...], wtb_ref[...],
                        preferred_element_type=jnp.float32)
    out_ref[...] = out + bias_ref[...]


def _lin2_call(a, b, wta, wtb, bias, out_dim):
    nb = N // R
    return pl.pallas_call(
        _lin2_body,
        grid=(nb,),
        in_specs=[
            pl.BlockSpec((R, D), lambda i: (i, 0)),
            pl.BlockSpec((R, D), lambda i: (i, 0)),
            pl.BlockSpec((D, out_dim), lambda i: (0, 0)),
            pl.BlockSpec((D, out_dim), lambda i: (0, 0)),
            pl.BlockSpec((1, out_dim), lambda i: (0, 0)),
        ],
        out_specs=pl.BlockSpec((R, out_dim), lambda i: (i, 0)),
        out_shape=jax.ShapeDtypeStruct((N, out_dim), jnp.float32),
    )(a, b, wta, wtb, bias)


def _make_scatter(KN):
    mesh = plsc.VectorSubcoreMesh(core_axis_name="c", subcore_axis_name="s")
    ept = E // NS
    nbatch = ept // EB
    rpt = N_PAD // NS

    @functools.partial(
        pl.kernel,
        out_type=[
            jax.ShapeDtypeStruct((N_PAD, H), jnp.float32),
            jax.ShapeDtypeStruct((N_PAD, H), jnp.float32),
        ],
        mesh=mesh,
        scratch_types=[
            pltpu.VMEM((ept,), jnp.int32),
            pltpu.VMEM((nbatch, EB), jnp.int32),
            pltpu.VMEM((EB, H), jnp.float32),
            pltpu.VMEM((EB, H), jnp.float32),
            pltpu.VMEM_SHARED((N_PAD, H), jnp.float32),
            pltpu.SemaphoreType.DMA,
            pltpu.SemaphoreType.DMA,
            pltpu.SemaphoreType.DMA,
            pltpu.SemaphoreType.DMA,
            pltpu.SemaphoreType.DMA,
        ],
    )
    def scatter_k(wh_lo, wh_hi, gidx_hbm, dst_hbm, zeros_hbm, out_lo, out_hi,
                  gidx_v, dst_v, rows0, rows1, acc, gsem0, gsem1, ssem0,
                  ssem1, isem):
        c = lax.axis_index("c")
        s = lax.axis_index("s")
        icp0 = pltpu.async_copy(gidx_hbm.at[s], gidx_v, isem)
        icp1 = pltpu.async_copy(dst_hbm.at[s], dst_v, isem)
        pltpu.sync_copy(zeros_hbm.at[pl.ds(s * rpt, rpt)],
                        acc.at[pl.ds(s * rpt, rpt)])
        icp0.wait()
        icp1.wait()
        plsc.subcore_barrier()

        rows = (rows0, rows1)
        gsem = (gsem0, gsem1)
        ssem = (ssem0, ssem1)

        def run(wh):
            def gather_start(i, b):
                pltpu.async_copy(wh.at[gidx_v.at[pl.ds(i * EB, EB)]],
                                 rows[b], gsem[b])

            def gather_wait(i, b):
                pltpu.make_async_copy(wh.at[gidx_v.at[pl.ds(i * EB, EB)]],
                                      rows[b], gsem[b]).wait()

            def scat_start(i, b):
                pltpu.async_copy(rows[b], acc.at[dst_v.at[i]], ssem[b],
                                 add=True)

            def scat_wait(i, b):
                pltpu.make_async_copy(rows[b], acc.at[dst_v.at[i]],
                                      ssem[b]).wait()

            gather_start(0, 0)

            @pl.loop(0, nbatch - 1, step=2)
            def _(g):
                gather_wait(g, 0)

                @pl.when(g > 0)
                def _():
                    scat_wait(g - 1, 1)

                gather_start(g + 1, 1)
                scat_start(g, 0)

                gather_wait(g + 1, 1)
                scat_wait(g, 0)
                gather_start(g + 2, 0)
                scat_start(g + 1, 1)

            gather_wait(nbatch - 1, 0)
            scat_wait(nbatch - 2, 1)
            pltpu.sync_copy(rows[0], acc.at[dst_v.at[nbatch - 1]], add=True)

        @pl.when(c == 0)
        def _():
            run(wh_lo)

        @pl.when(c == 1)
        def _():
            run(wh_hi)

        plsc.subcore_barrier()

        ob = s * rpt

        @pl.when(c == 0)
        def _():
            pltpu.sync_copy(acc.at[pl.ds(ob, rpt)], out_lo.at[pl.ds(ob, rpt)])

        @pl.when(c == 1)
        def _():
            pltpu.sync_copy(acc.at[pl.ds(ob, rpt)], out_hi.at[pl.ds(ob, rpt)])

    return scatter_k


_scatter_ast = _make_scatter(N)
_scatter_cpg = _make_scatter(3 * N)


def _conv(h, gidx, dst, zeros, wT, b, wihT, bih, whhT, bhh, scatter, K):
    nb = (E // NS) // EB
    gidx = gidx.reshape(NS, E // NS)
    dst = dst.reshape(NS, nb, EB)
    for _ in range(N_STEPS):
        lo, hi = _wh_call(h, wT, b, K)
        alo, ahi = scatter(lo, hi, gidx, dst, zeros)
        h = _gru_call(alo[:N], ahi[:N], h, wihT, bih, whhT, bhh)
    return h


def kernel(x, ast_edge_index, cpg_edge_index, cpg_etypes, ast_W, ast_b,
           ast_Wih, ast_bih, ast_Whh, ast_bhh, cpg_Ws, cpg_bs, cpg_Wih,
           cpg_bih, cpg_Whh, cpg_bhh, fn1_W, fn1_b, fn2_W, fn2_b):
    zeros = jnp.zeros((N_PAD, H), jnp.float32)

    gidx_a = ast_edge_index[0].astype(jnp.int32)
    dst_a = ast_edge_index[1].astype(jnp.int32)
    h_ast = _conv(
        x, gidx_a, dst_a, zeros,
        jnp.transpose(ast_W)[None], ast_b[None, None],
        jnp.transpose(ast_Wih), ast_bih[None], jnp.transpose(ast_Whh),
        ast_bhh[None], _scatter_ast, 1)

    fn1_WT = jnp.transpose(fn1_W)
    hiddens = _lin2_call(h_ast, x, fn1_WT[:D], fn1_WT[D:], fn1_b[None], D)

    gidx_c = (cpg_etypes.astype(jnp.int32) * N
              + cpg_edge_index[0].astype(jnp.int32))
    dst_c = cpg_edge_index[1].astype(jnp.int32)
    h_cpg = _conv(
        hiddens, gidx_c, dst_c, zeros,
        jnp.transpose(cpg_Ws, (0, 2, 1)), cpg_bs[:, None],
        jnp.transpose(cpg_Wih), cpg_bih[None], jnp.transpose(cpg_Whh),
        cpg_bhh[None], _scatter_cpg, 3)

    fn2_WT = jnp.transpose(fn2_W)
    logits = _lin2_call(h_cpg, hiddens, fn2_WT[:D], fn2_WT[D:], fn2_b[None], D)
    return logits
